# baseline (speedup 1.0000x reference)
import functools

import jax
import jax.numpy as jnp
from jax import lax
from jax.experimental import pallas as pl
from jax.experimental.pallas import tpu as pltpu
from jax.experimental.pallas import tpu_sc as plsc

B = 64
N = 4096
K_SEL = 128
L = 16
NV = N // L
NC = 2
NS = 16
NW = NC * NS
ROWS_PER_W = B // NW

_I32_MIN = -2147483648


def _lane():
    return lax.broadcasted_iota(jnp.int32, (L,), 0)


def _pc(m):
    return plsc.all_reduce_population_count(m)


def _take(v, i):
    return jnp.take_along_axis(v, i, axis=0, mode="promise_in_bounds")


def _rcr(v):
    return lax.rev(plsc.cumsum(lax.rev(v, (0,))), (0,))


def _key_of(x):
    bits = plsc.bitcast(x, jnp.int32)
    bits = jnp.where(bits == _I32_MIN, 0, bits)
    return jnp.where(bits >= 0, bits, bits ^ jnp.int32(0x7FFFFFFF))


def _select_bucket(hist, kk):
    lane = _lane()
    vs = [hist[pl.ds(i * L, L)] for i in range(16)]
    while len(vs) > 1:
        vs = [vs[i] + vs[i + 1] for i in range(0, len(vs), 2)]
    ct = vs[0]
    suf_c = _rcr(ct)
    jc = jnp.maximum(_pc(suf_c >= kk) - 1, 0)
    cnt_above = _take(suf_c, jc) - _take(ct, jc)
    cvec = plsc.load_gather(hist, [lane * L + jc])
    wsuf = _rcr(cvec) + cnt_above
    fm = wsuf >= kk
    ib = jnp.maximum(_pc(fm) - 1, 0)
    wib = _take(wsuf, ib)
    cib = _take(cvec, ib)
    b = jc * L + ib
    kn = kk - (wib - cib)
    return b, kn, cib


def _zero_hist(hist):
    z = jnp.zeros((L,), jnp.int32)
    for j in range(16):
        hist[pl.ds(j * L, L)] = z


def _sc_body(x_hbm, out_hbm, xv, kbuf, cbuf, pbuf, cb2, cb3, hist, outv):
    wid = lax.axis_index("s") * NC + lax.axis_index("c")
    base = wid * ROWS_PER_W
    pltpu.sync_copy(x_hbm.at[pl.ds(base, ROWS_PER_W)], xv)

    lane = _lane()
    ones = jnp.ones((L,), jnp.int32)
    zero_splat = jnp.zeros((L,), jnp.int32)

    def row_body(r, _):
        _zero_hist(hist)

        @plsc.parallel_loop(0, NV, unroll=4)
        def _pass_a(i):
            key = _key_of(xv[r, pl.ds(i * L, L)])
            kbuf[pl.ds(i * L, L)] = key
            t = key >> 24
            idx_t = ((t & 0xF) << 4) | (((t >> 4) & 0xF) ^ 0x8)
            plsc.addupdate_scatter(hist, [idx_t], ones)

        kk = jnp.full((L,), K_SEL, jnp.int32)
        b0, kk, mb = _select_bucket(hist, kk)
        p0 = b0 - 128

        @plsc.parallel_loop(0, NV, carry=zero_splat)
        def _extract0(i, off):
            key = kbuf[pl.ds(i * L, L)]
            t = key >> 24
            outv[r, pl.ds(i * L, L)] = jnp.where(
                t > p0, jnp.float32(1.0), jnp.float32(0.0))
            mm = t == p0
            pos = off + plsc.cumsum(mm.astype(jnp.int32)) - 1
            plsc.store_scatter(cbuf, [pos], key, mask=mm)
            plsc.store_scatter(pbuf, [pos], i * L + lane, mask=mm)
            return off + _pc(mm)

        m0 = jnp.max(mb)
        ncand = (m0 + (L - 1)) // L
        m = m0

        def refine_stage(src_ref, dst_ref, m, kk, shift):
            small = m <= L
            nvec = jnp.where(small, 0, (m + (L - 1)) // L)
            _zero_hist(hist)

            @plsc.parallel_loop(0, nvec)
            def _hpass(i):
                kv = src_ref[pl.ds(i * L, L)]
                valid = (i * L + lane) < m
                t = kv >> shift
                idx_t = ((t & 0xF) << 4) | ((t >> 4) & 0xF)
                plsc.addupdate_scatter(hist, [idx_t], ones, mask=valid)

            b, kn, mbv = _select_bucket(hist, kk)
            dst_ref[pl.ds(0, L)] = src_ref[pl.ds(0, L)]

            @plsc.parallel_loop(0, nvec, carry=zero_splat)
            def _epass(i, off):
                kv = src_ref[pl.ds(i * L, L)]
                valid = (i * L + lane) < m
                mm = jnp.logical_and(((kv >> shift) & 0xFF) == b, valid)
                pos = off + plsc.cumsum(mm.astype(jnp.int32)) - 1
                plsc.store_scatter(dst_ref, [pos], kv, mask=mm)
                return off + _pc(mm)

            m2 = jnp.where(small, m, jnp.max(mbv))
            kk2 = jnp.where(small, kk, kn)
            return m2, kk2

        m, kk = refine_stage(cbuf, cb2, m, kk, 16)
        m, kk = refine_stage(cb2, cb3, m, kk, 8)
        m, kk = refine_stage(cb3, cb2, m, kk, 0)

        kv = cb2[pl.ds(0, L)]
        kvm = jnp.where(lane < m, kv, _I32_MIN)
        skey, _ = plsc.sort_key_val(kvm, kvm, descending=True)
        t_sorted = _take(skey, kk - 1)
        t_first = _take(kv, zero_splat)
        tkey = jnp.where(m <= L, t_sorted, t_first)

        @plsc.parallel_loop(0, ncand)
        def _fixup(i):
            kv = cbuf[pl.ds(i * L, L)]
            pv = pbuf[pl.ds(i * L, L)]
            valid = (i * L + lane) < m0
            o = jnp.where(kv >= tkey, jnp.float32(1.0), jnp.float32(0.0))
            plsc.store_scatter(outv, [zero_splat + r, pv], o, mask=valid)

        return 0

    lax.fori_loop(0, ROWS_PER_W, row_body, 0)
    pltpu.sync_copy(outv, out_hbm.at[pl.ds(base, ROWS_PER_W)])


@functools.partial(
    pl.kernel,
    out_type=jax.ShapeDtypeStruct((B, N), jnp.float32),
    mesh=plsc.VectorSubcoreMesh(core_axis_name="c", subcore_axis_name="s",
                                num_cores=NC, num_subcores=NS),
    compiler_params=pltpu.CompilerParams(needs_layout_passes=False,
                                         use_tc_tiling_on_sc=False),
    scratch_types=[
        pltpu.VMEM((ROWS_PER_W, N), jnp.float32),
        pltpu.VMEM((N,), jnp.int32),
        pltpu.VMEM((N,), jnp.int32),
        pltpu.VMEM((N,), jnp.int32),
        pltpu.VMEM((N,), jnp.int32),
        pltpu.VMEM((N,), jnp.int32),
        pltpu.VMEM((256,), jnp.int32),
        pltpu.VMEM((ROWS_PER_W, N), jnp.float32),
    ],
)
def _sc_topk(x_hbm, out_hbm, xv, kbuf, cbuf, pbuf, cb2, cb3, hist, outv):
    _sc_body(x_hbm, out_hbm, xv, kbuf, cbuf, pbuf, cb2, cb3, hist, outv)


@jax.jit
def kernel(logits):
    x = logits.reshape(B, N)
    out = _sc_topk(x)
    return out.reshape(B, N, 1)


# speedup vs baseline: 1.0205x; 1.0205x over previous
import functools

import jax
import jax.numpy as jnp
from jax import lax
from jax.experimental import pallas as pl
from jax.experimental.pallas import tpu as pltpu
from jax.experimental.pallas import tpu_sc as plsc

B = 64
N = 4096
K_SEL = 128
L = 16
NV = N // L
NC = 2
NS = 16
NW = NC * NS
ROWS_PER_W = B // NW

_I32_MIN = -2147483648


def _lane():
    return lax.broadcasted_iota(jnp.int32, (L,), 0)


def _pc(m):
    return plsc.all_reduce_population_count(m)


def _take(v, i):
    return jnp.take_along_axis(v, i, axis=0, mode="promise_in_bounds")


def _rcr(v):
    return lax.rev(plsc.cumsum(lax.rev(v, (0,))), (0,))


def _key_of(x):
    bits = plsc.bitcast(x, jnp.int32)
    bits = jnp.where(bits == _I32_MIN, 0, bits)
    return jnp.where(bits >= 0, bits, bits ^ jnp.int32(0x7FFFFFFF))


def _select_bucket(hist, kk, histb=None):
    lane = _lane()
    vs = [hist[pl.ds(i * L, L)] for i in range(16)]
    if histb is not None:
        vs += [histb[pl.ds(i * L, L)] for i in range(16)]
    while len(vs) > 1:
        vs = [vs[i] + vs[i + 1] for i in range(0, len(vs), 2)]
    ct = vs[0]
    suf_c = _rcr(ct)
    jc = jnp.maximum(_pc(suf_c >= kk) - 1, 0)
    cnt_above = _take(suf_c, jc) - _take(ct, jc)
    cvec = plsc.load_gather(hist, [lane * L + jc])
    if histb is not None:
        cvec = cvec + plsc.load_gather(histb, [lane * L + jc])
    wsuf = _rcr(cvec) + cnt_above
    fm = wsuf >= kk
    ib = jnp.maximum(_pc(fm) - 1, 0)
    wib = _take(wsuf, ib)
    cib = _take(cvec, ib)
    b = jc * L + ib
    kn = kk - (wib - cib)
    return b, kn, cib


def _zero_hist(hist):
    z = jnp.zeros((L,), jnp.int32)
    for j in range(16):
        hist[pl.ds(j * L, L)] = z


def _sc_body(x_hbm, out_hbm, xv, kbuf, cbuf, cb2, cb3, hist, histb, outv):
    wid = lax.axis_index("s") * NC + lax.axis_index("c")
    base = wid * ROWS_PER_W
    pltpu.sync_copy(x_hbm.at[pl.ds(base, ROWS_PER_W)], xv)

    lane = _lane()
    ones = jnp.ones((L,), jnp.int32)
    zero_splat = jnp.zeros((L,), jnp.int32)

    def row_body(r, _):
        _zero_hist(hist)
        _zero_hist(histb)

        @plsc.parallel_loop(0, NV // 2, unroll=2)
        def _pass_a(i):
            key_a = _key_of(xv[r, pl.ds(2 * i * L, L)])
            kbuf[pl.ds(2 * i * L, L)] = key_a
            ta = key_a >> 24
            idx_a = ((ta & 0xF) << 4) | (((ta >> 4) & 0xF) ^ 0x8)
            plsc.addupdate_scatter(hist, [idx_a], ones)
            key_b = _key_of(xv[r, pl.ds((2 * i + 1) * L, L)])
            kbuf[pl.ds((2 * i + 1) * L, L)] = key_b
            tb = key_b >> 24
            idx_b = ((tb & 0xF) << 4) | (((tb >> 4) & 0xF) ^ 0x8)
            plsc.addupdate_scatter(histb, [idx_b], ones)

        kk = jnp.full((L,), K_SEL, jnp.int32)
        b0, kk, mb = _select_bucket(hist, kk, histb)
        p0 = b0 - 128

        @plsc.parallel_loop(0, NV, carry=zero_splat)
        def _extract0(i, off):
            key = kbuf[pl.ds(i * L, L)]
            mm = (key >> 24) == p0
            pos = off + plsc.cumsum(mm.astype(jnp.int32)) - 1
            plsc.store_scatter(cbuf, [pos], key, mask=mm)
            return off + _pc(mm)

        m = jnp.max(mb)

        def refine_stage(src_ref, dst_ref, m, kk, shift):
            small = m <= L
            nvec = jnp.where(small, 0, (m + (L - 1)) // L)
            _zero_hist(hist)

            @plsc.parallel_loop(0, nvec)
            def _hpass(i):
                kv = src_ref[pl.ds(i * L, L)]
                valid = (i * L + lane) < m
                t = kv >> shift
                idx_t = ((t & 0xF) << 4) | ((t >> 4) & 0xF)
                plsc.addupdate_scatter(hist, [idx_t], ones, mask=valid)

            b, kn, mbv = _select_bucket(hist, kk)
            dst_ref[pl.ds(0, L)] = src_ref[pl.ds(0, L)]

            @plsc.parallel_loop(0, nvec, carry=zero_splat)
            def _epass(i, off):
                kv = src_ref[pl.ds(i * L, L)]
                valid = (i * L + lane) < m
                mm = jnp.logical_and(((kv >> shift) & 0xFF) == b, valid)
                pos = off + plsc.cumsum(mm.astype(jnp.int32)) - 1
                plsc.store_scatter(dst_ref, [pos], kv, mask=mm)
                return off + _pc(mm)

            m2 = jnp.where(small, m, jnp.max(mbv))
            kk2 = jnp.where(small, kk, kn)
            return m2, kk2

        m, kk = refine_stage(cbuf, cb2, m, kk, 16)
        m, kk = refine_stage(cb2, cb3, m, kk, 8)
        m, kk = refine_stage(cb3, cb2, m, kk, 0)

        kv = cb2[pl.ds(0, L)]
        kvm = jnp.where(lane < m, kv, _I32_MIN)
        skey, _ = plsc.sort_key_val(kvm, kvm, descending=True)
        t_sorted = _take(skey, kk - 1)
        t_first = _take(kv, zero_splat)
        tkey = jnp.where(m <= L, t_sorted, t_first)

        @plsc.parallel_loop(0, NV, unroll=4)
        def _mask_pass(i):
            kv = kbuf[pl.ds(i * L, L)]
            outv[r, pl.ds(i * L, L)] = jnp.where(
                kv >= tkey, jnp.float32(1.0), jnp.float32(0.0))

        return 0

    lax.fori_loop(0, ROWS_PER_W, row_body, 0)
    pltpu.sync_copy(outv, out_hbm.at[pl.ds(base, ROWS_PER_W)])


@functools.partial(
    pl.kernel,
    out_type=jax.ShapeDtypeStruct((B, N), jnp.float32),
    mesh=plsc.VectorSubcoreMesh(core_axis_name="c", subcore_axis_name="s",
                                num_cores=NC, num_subcores=NS),
    compiler_params=pltpu.CompilerParams(needs_layout_passes=False,
                                         use_tc_tiling_on_sc=False),
    scratch_types=[
        pltpu.VMEM((ROWS_PER_W, N), jnp.float32),
        pltpu.VMEM((N,), jnp.int32),
        pltpu.VMEM((N,), jnp.int32),
        pltpu.VMEM((N,), jnp.int32),
        pltpu.VMEM((N,), jnp.int32),
        pltpu.VMEM((256,), jnp.int32),
        pltpu.VMEM((256,), jnp.int32),
        pltpu.VMEM((ROWS_PER_W, N), jnp.float32),
    ],
)
def _sc_topk(x_hbm, out_hbm, xv, kbuf, cbuf, cb2, cb3, hist, histb, outv):
    _sc_body(x_hbm, out_hbm, xv, kbuf, cbuf, cb2, cb3, hist, histb, outv)


@jax.jit
def kernel(logits):
    x = logits.reshape(B, N)
    out = _sc_topk(x)
    return out.reshape(B, N, 1)
